# baseline (speedup 1.0000x reference)
import jax
import jax.numpy as jnp
from jax.experimental import pallas as pl
from jax.experimental.pallas import tpu as pltpu

_NUM_CB = 1024
_DIM = 32
_EPS = 1e-07
_N = 65536
_BLK = 512
_GRID = _N // _BLK


def _vq_body(x_ref, cb_ref, xq_ref, oh_ref, perp_ref, counts_ref):
    i = pl.program_id(0)
    x = x_ref[...]
    cb = cb_ref[...]
    mm = jax.lax.dot_general(x, cb, (((1,), (1,)), ((), ())),
                             preferred_element_type=jnp.float32)
    xsq = jnp.sum(x * x, axis=-1, keepdims=True)
    cbsq = jnp.sum(cb * cb, axis=-1)
    dist = (xsq + cbsq) - 2.0 * mm
    minval = jnp.min(dist, axis=-1, keepdims=True)
    oh = jnp.where(dist == minval, 1.0, 0.0)
    oh_ref[...] = oh
    xq_ref[...] = jax.lax.dot_general(oh, cb, (((1,), (0,)), ((), ())),
                                      preferred_element_type=jnp.float32)
    ones_row = jnp.ones((1, _BLK), dtype=jnp.float32)
    bc = jax.lax.dot_general(ones_row, oh, (((1,), (0,)), ((), ())),
                             preferred_element_type=jnp.float32)

    @pl.when(i == 0)
    def _():
        counts_ref[...] = bc

    @pl.when(i > 0)
    def _():
        counts_ref[...] = counts_ref[...] + bc

    @pl.when(i == _GRID - 1)
    def _():
        cnt = counts_ref[...]
        prob = cnt / jnp.sum(cnt)
        perp = jnp.exp(-jnp.sum(prob * jnp.log(prob + _EPS)))
        perp_ref[...] = jnp.full((1, 1), perp, dtype=jnp.float32)


def kernel(flat_x, codebook):
    xq, oh, perp = pl.pallas_call(
        _vq_body,
        grid=(_GRID,),
        in_specs=[
            pl.BlockSpec((_BLK, _DIM), lambda i: (i, 0)),
            pl.BlockSpec((_NUM_CB, _DIM), lambda i: (0, 0)),
        ],
        out_specs=[
            pl.BlockSpec((_BLK, _DIM), lambda i: (i, 0)),
            pl.BlockSpec((_BLK, _NUM_CB), lambda i: (i, 0)),
            pl.BlockSpec((1, 1), lambda i: (0, 0)),
        ],
        out_shape=[
            jax.ShapeDtypeStruct((_N, _DIM), jnp.float32),
            jax.ShapeDtypeStruct((_N, _NUM_CB), jnp.float32),
            jax.ShapeDtypeStruct((1, 1), jnp.float32),
        ],
        scratch_shapes=[pltpu.VMEM((1, _NUM_CB), jnp.float32)],
    )(flat_x, codebook)
    return (xq, oh, perp[0, 0])


# speedup vs baseline: 1.1906x; 1.1906x over previous
import jax
import jax.numpy as jnp
from jax.experimental import pallas as pl
from jax.experimental.pallas import tpu as pltpu

_NUM_CB = 1024
_DIM = 32
_EPS = 1e-07
_N = 65536
_BLK = 2048
_GRID = _N // _BLK


def _vq_body(x_ref, cb_ref, xq_ref, oh_ref, perp_ref, counts_ref):
    i = pl.program_id(0)
    x = x_ref[...]
    cb = cb_ref[...]
    mm = jax.lax.dot_general(x, cb, (((1,), (1,)), ((), ())),
                             preferred_element_type=jnp.float32)
    xsq = jnp.sum(x * x, axis=-1, keepdims=True)
    cbsq = jnp.sum(cb * cb, axis=-1)
    dist = (xsq + cbsq) - 2.0 * mm
    minval = jnp.min(dist, axis=-1, keepdims=True)
    oh = jnp.where(dist == minval, 1.0, 0.0)
    oh_ref[...] = oh
    xq_ref[...] = jax.lax.dot_general(oh, cb, (((1,), (0,)), ((), ())),
                                      preferred_element_type=jnp.float32)
    ones_row = jnp.ones((1, _BLK), dtype=jnp.float32)
    bc = jax.lax.dot_general(ones_row, oh, (((1,), (0,)), ((), ())),
                             preferred_element_type=jnp.float32)

    @pl.when(i == 0)
    def _():
        counts_ref[...] = bc

    @pl.when(i > 0)
    def _():
        counts_ref[...] = counts_ref[...] + bc

    @pl.when(i == _GRID - 1)
    def _():
        cnt = counts_ref[...]
        prob = cnt / jnp.sum(cnt)
        perp = jnp.exp(-jnp.sum(prob * jnp.log(prob + _EPS)))
        perp_ref[...] = jnp.full((1, 1), perp, dtype=jnp.float32)


def kernel(flat_x, codebook):
    xq, oh, perp = pl.pallas_call(
        _vq_body,
        grid=(_GRID,),
        in_specs=[
            pl.BlockSpec((_BLK, _DIM), lambda i: (i, 0)),
            pl.BlockSpec((_NUM_CB, _DIM), lambda i: (0, 0)),
        ],
        out_specs=[
            pl.BlockSpec((_BLK, _DIM), lambda i: (i, 0)),
            pl.BlockSpec((_BLK, _NUM_CB), lambda i: (i, 0)),
            pl.BlockSpec((1, 1), lambda i: (0, 0)),
        ],
        out_shape=[
            jax.ShapeDtypeStruct((_N, _DIM), jnp.float32),
            jax.ShapeDtypeStruct((_N, _NUM_CB), jnp.float32),
            jax.ShapeDtypeStruct((1, 1), jnp.float32),
        ],
        scratch_shapes=[pltpu.VMEM((1, _NUM_CB), jnp.float32)],
    )(flat_x, codebook)
    return (xq, oh, perp[0, 0])


# speedup vs baseline: 1.2628x; 1.0606x over previous
import jax
import jax.numpy as jnp
from jax.experimental import pallas as pl
from jax.experimental.pallas import tpu as pltpu

_NUM_CB = 1024
_DIM = 32
_EPS = 1e-07
_N = 65536
_BLK = 2048
_GRID = _N // _BLK


def _vq_body(x_ref, cb_ref, xq_ref, oh_ref, perp_ref, counts_ref):
    i = pl.program_id(0)
    x = x_ref[...]
    cb = cb_ref[...]
    cbsq = jnp.sum(cb * cb, axis=-1, keepdims=True)
    cb_aug = jnp.concatenate([cb * -2.0, cbsq], axis=1)
    x_aug = jnp.concatenate([x, jnp.ones((_BLK, 1), jnp.float32)], axis=1)
    dist = jax.lax.dot_general(x_aug, cb_aug, (((1,), (1,)), ((), ())),
                               preferred_element_type=jnp.float32)
    minval = jnp.min(dist, axis=-1, keepdims=True)
    oh = jnp.where(dist == minval, 1.0, 0.0)
    oh_ref[...] = oh
    xq_ref[...] = jax.lax.dot_general(oh, cb, (((1,), (0,)), ((), ())),
                                      preferred_element_type=jnp.float32)
    ones_row = jnp.ones((1, _BLK), dtype=jnp.float32)
    bc = jax.lax.dot_general(ones_row, oh, (((1,), (0,)), ((), ())),
                             preferred_element_type=jnp.float32)

    @pl.when(i == 0)
    def _():
        counts_ref[...] = bc

    @pl.when(i > 0)
    def _():
        counts_ref[...] = counts_ref[...] + bc

    @pl.when(i == _GRID - 1)
    def _():
        cnt = counts_ref[...]
        prob = cnt / jnp.sum(cnt)
        perp = jnp.exp(-jnp.sum(prob * jnp.log(prob + _EPS)))
        perp_ref[...] = jnp.full((1, 1), perp, dtype=jnp.float32)


def kernel(flat_x, codebook):
    xq, oh, perp = pl.pallas_call(
        _vq_body,
        grid=(_GRID,),
        in_specs=[
            pl.BlockSpec((_BLK, _DIM), lambda i: (i, 0)),
            pl.BlockSpec((_NUM_CB, _DIM), lambda i: (0, 0)),
        ],
        out_specs=[
            pl.BlockSpec((_BLK, _DIM), lambda i: (i, 0)),
            pl.BlockSpec((_BLK, _NUM_CB), lambda i: (i, 0)),
            pl.BlockSpec((1, 1), lambda i: (0, 0)),
        ],
        out_shape=[
            jax.ShapeDtypeStruct((_N, _DIM), jnp.float32),
            jax.ShapeDtypeStruct((_N, _NUM_CB), jnp.float32),
            jax.ShapeDtypeStruct((1, 1), jnp.float32),
        ],
        scratch_shapes=[pltpu.VMEM((1, _NUM_CB), jnp.float32)],
    )(flat_x, codebook)
    return (xq, oh, perp[0, 0])
